# baseline (speedup 1.0000x reference)
import jax
import jax.numpy as jnp
from jax import lax
from jax.experimental import pallas as pl
from jax.experimental.pallas import tpu as pltpu

_NS = 4


def _part_argmax(blk):
    m = blk.shape[0]
    mx = jnp.max(blk, axis=0)
    iota = lax.broadcasted_iota(jnp.int32, blk.shape, 0)
    idx = jnp.min(jnp.where(blk == mx[None, :], iota, m), axis=0)
    return mx, idx


def _argmax_body(*refs):
    o_ref = refs[-1]
    x_refs = refs[:-1]
    mh = x_refs[0].shape[1]
    mx, idx = _part_argmax(x_refs[0][0])
    for i in range(1, len(x_refs)):
        mx_i, idx_i = _part_argmax(x_refs[i][0])
        keep = mx >= mx_i
        mx = jnp.where(keep, mx, mx_i)
        idx = jnp.where(keep, idx, idx_i + i * mh)
    o_ref[0, 0] = idx


def kernel(x):
    B, M, N = x.shape
    MH = M // _NS
    specs = [
        pl.BlockSpec((1, MH, N), lambda b, i=i: (b, i, 0)) for i in range(_NS)
    ]
    out = pl.pallas_call(
        _argmax_body,
        grid=(B,),
        in_specs=specs,
        out_specs=pl.BlockSpec((1, 1, N), lambda b: (b, 0, 0)),
        out_shape=jax.ShapeDtypeStruct((B, 1, N), jnp.int32),
    )(*([x] * _NS))
    return out.reshape(B, N)
